# speedup vs baseline: 2.9834x; 2.9834x over previous
import functools

import jax
import jax.numpy as jnp
from jax import lax
from jax.experimental import pallas as pl
from jax.experimental.pallas import tpu as pltpu
from jax.experimental.pallas import tpu_sc as plsc

_CHUNK = 80
_NSLOT = 5


@functools.lru_cache(maxsize=None)
def _build_gather(n_rows, d):
    info = plsc.get_sparse_core_info()
    nw = info.num_cores * info.num_subcores
    assert n_rows % nw == 0
    b_per_w = n_rows // nw
    assert b_per_w % _CHUNK == 0
    n_chunks = b_per_w // _CHUNK
    assert n_chunks % _NSLOT == 0
    n_outer = n_chunks // _NSLOT

    mesh = plsc.VectorSubcoreMesh(core_axis_name="c", subcore_axis_name="s")

    @functools.partial(
        pl.kernel,
        mesh=mesh,
        out_type=jax.ShapeDtypeStruct((n_rows, d), jnp.float32),
        scratch_types=[
            pltpu.VMEM((b_per_w,), jnp.int32),
            pltpu.VMEM((_NSLOT, _CHUNK, d), jnp.float32),
        ]
        + [pltpu.SemaphoreType.DMA] * (2 * _NSLOT),
    )
    def gather_k(table_hbm, idx_hbm, out_hbm, idx_v, buf, *sems):
        gsem = sems[:_NSLOT]
        wsem = sems[_NSLOT:]
        wid = lax.axis_index("s") * info.num_cores + lax.axis_index("c")
        base = wid * b_per_w

        pltpu.sync_copy(idx_hbm.at[pl.ds(base, b_per_w)], idx_v)

        def gather_desc(c, j):
            return pltpu.make_async_copy(
                table_hbm.at[idx_v.at[pl.ds(c * _CHUNK, _CHUNK)]],
                buf.at[j],
                gsem[j],
            )

        def write_desc(c, j):
            return pltpu.make_async_copy(
                buf.at[j],
                out_hbm.at[pl.ds(base + c * _CHUNK, _CHUNK)],
                wsem[j],
            )

        for j in range(_NSLOT):
            gather_desc(j, j).start()

        def body(p, carry):
            for j in range(_NSLOT):
                c = p * _NSLOT + j
                gather_desc(c, j).wait()
                write_desc(c, j).start()

            @pl.when(p + 1 < n_outer)
            def _():
                for j in range(_NSLOT):
                    write_desc(p * _NSLOT + j, j).wait()
                    gather_desc((p + 1) * _NSLOT + j, j).start()

            return carry

        lax.fori_loop(0, n_outer, body, 0)

        for j in range(_NSLOT):
            write_desc((n_outer - 1) * _NSLOT + j, j).wait()

    return gather_k


def kernel(logits, G):
    n, d = logits.shape
    k = G.shape[1]
    idx = G.reshape(-1).astype(jnp.int32)
    out = _build_gather(n * k, d)(logits, idx)
    return out.reshape(n, k * d)


# speedup vs baseline: 5.4916x; 1.8407x over previous
import functools

import jax
import jax.numpy as jnp
from jax import lax
from jax.experimental import pallas as pl
from jax.experimental.pallas import tpu as pltpu
from jax.experimental.pallas import tpu_sc as plsc

_CHUNK = 80
_NSLOT = 5


@functools.lru_cache(maxsize=None)
def _build_gather(n_rows, d):
    info = plsc.get_sparse_core_info()
    nw = info.num_cores * info.num_subcores
    assert n_rows % nw == 0
    b_per_w = n_rows // nw
    assert b_per_w % _CHUNK == 0
    n_chunks = b_per_w // _CHUNK
    assert n_chunks % _NSLOT == 0
    n_outer = n_chunks // _NSLOT

    mesh = plsc.VectorSubcoreMesh(core_axis_name="c", subcore_axis_name="s")

    @functools.partial(
        pl.kernel,
        mesh=mesh,
        out_type=jax.ShapeDtypeStruct((n_rows, d), jnp.float32),
        scratch_types=[
            pltpu.VMEM((b_per_w,), jnp.int32),
            pltpu.VMEM((_NSLOT, _CHUNK, d), jnp.float32),
        ]
        + [pltpu.SemaphoreType.DMA] * (2 * _NSLOT),
    )
    def gather_k(table_hbm, idx_hbm, out_hbm, idx_v, buf, *sems):
        gsem = sems[:_NSLOT]
        wsem = sems[_NSLOT:]
        wid = lax.axis_index("s") * info.num_cores + lax.axis_index("c")
        base = wid * b_per_w

        pltpu.sync_copy(idx_hbm.at[pl.ds(base, b_per_w)], idx_v)

        def gather_desc(c, j):
            return pltpu.make_async_copy(
                table_hbm.at[idx_v.at[pl.ds(c * _CHUNK, _CHUNK)]],
                buf.at[j],
                gsem[j],
            )

        def write_desc(c, j):
            return pltpu.make_async_copy(
                buf.at[j],
                out_hbm.at[pl.ds(base + c * _CHUNK, _CHUNK)],
                wsem[j],
            )

        for j in range(_NSLOT):
            gather_desc(j, j).start()

        def body(p, carry):
            for j in range(_NSLOT):
                c = p * _NSLOT + j
                gather_desc(c, j).wait()
                write_desc(c, j).start()

            @pl.when(p + 1 < n_outer)
            def _():
                for j in range(_NSLOT):
                    write_desc(p * _NSLOT + j, j).wait()
                    gather_desc((p + 1) * _NSLOT + j, j).start()

            return carry

        lax.fori_loop(0, n_outer, body, 0)

        for j in range(_NSLOT):
            write_desc((n_outer - 1) * _NSLOT + j, j).wait()

    return gather_k


def kernel(logits, G):
    n, d = logits.shape
    k = G.shape[1]
    idx = G.astype(jnp.int32).reshape(n // 8, 8, k).transpose(0, 2, 1).reshape(-1)
    out = _build_gather(n * k, d)(logits, idx)
    return out.reshape(n // 8, k, 8, d).transpose(0, 2, 1, 3).reshape(n, k * d)


# speedup vs baseline: 6.4163x; 1.1684x over previous
import functools

import jax
import jax.numpy as jnp
from jax import lax
from jax.experimental import pallas as pl
from jax.experimental.pallas import tpu as pltpu
from jax.experimental.pallas import tpu_sc as plsc

_CHUNK = 80
_NSLOT = 5
_LANES = 16


@functools.lru_cache(maxsize=None)
def _build_gather(n, k, d):
    n_rows = n * k
    slab = 8 * k
    info = plsc.get_sparse_core_info()
    nw = info.num_cores * info.num_subcores
    assert n_rows % nw == 0
    b_per_w = n_rows // nw
    assert b_per_w % (_CHUNK * _NSLOT) == 0 and _CHUNK % _LANES == 0
    assert slab & (slab - 1) == 0
    n_chunks = b_per_w // _CHUNK
    n_outer = n_chunks // _NSLOT
    g_load = (b_per_w // slab + 2) * slab

    mesh = plsc.VectorSubcoreMesh(core_axis_name="c", subcore_axis_name="s")

    @functools.partial(
        pl.kernel,
        mesh=mesh,
        out_type=jax.ShapeDtypeStruct((n_rows, d), jnp.float32),
        scratch_types=[
            pltpu.VMEM((g_load,), jnp.int32),
            pltpu.VMEM((_NSLOT, _CHUNK), jnp.int32),
            pltpu.VMEM((_NSLOT, _CHUNK, d), jnp.float32),
        ]
        + [pltpu.SemaphoreType.DMA] * (2 * _NSLOT),
        compiler_params=pltpu.CompilerParams(needs_layout_passes=False),
    )
    def gather_k(table_hbm, idx_hbm, out_hbm, gsrc, idxb, buf, *sems):
        gsem = sems[:_NSLOT]
        wsem = sems[_NSLOT:]
        wid = lax.axis_index("s") * info.num_cores + lax.axis_index("c")
        p0 = wid * b_per_w
        off = jnp.minimum((p0 // slab) * slab, n_rows - g_load)
        pltpu.sync_copy(idx_hbm.at[pl.ds(off, g_load)], gsrc)

        def fill_idx(c, j):
            for g in range(_CHUNK // _LANES):
                p_vec = (p0 + c * _CHUNK + g * _LANES) + lax.iota(jnp.int32, _LANES)
                t = p_vec & (slab - 1)
                src = (p_vec - t - off) + (t & 7) * k + (t >> 3)
                idxb[j, pl.ds(g * _LANES, _LANES)] = plsc.load_gather(gsrc, [src])

        def gather_desc(j):
            return pltpu.make_async_copy(
                table_hbm.at[idxb.at[j]], buf.at[j], gsem[j])

        def write_desc(c, j):
            return pltpu.make_async_copy(
                buf.at[j],
                out_hbm.at[pl.ds(p0 + c * _CHUNK, _CHUNK)],
                wsem[j],
            )

        for j in range(_NSLOT):
            fill_idx(j, j)
            gather_desc(j).start()

        def body(p, carry):
            for j in range(_NSLOT):
                gather_desc(j).wait()
                write_desc(p * _NSLOT + j, j).start()

            @pl.when(p + 1 < n_outer)
            def _():
                for j in range(_NSLOT):
                    write_desc(p * _NSLOT + j, j).wait()
                    fill_idx((p + 1) * _NSLOT + j, j)
                    gather_desc(j).start()

            return carry

        lax.fori_loop(0, n_outer, body, 0)

        for j in range(_NSLOT):
            write_desc((n_outer - 1) * _NSLOT + j, j).wait()

    return gather_k


def kernel(logits, G):
    n, d = logits.shape
    k = G.shape[1]
    idx = G.astype(jnp.int32).reshape(-1)
    out = _build_gather(n, k, d)(logits, idx)
    return out.reshape(n // 8, k, 8, d).transpose(0, 2, 1, 3).reshape(n, k * d)


# speedup vs baseline: 6.4912x; 1.0117x over previous
import functools

import jax
import jax.numpy as jnp
from jax import lax
from jax.experimental import pallas as pl
from jax.experimental.pallas import tpu as pltpu
from jax.experimental.pallas import tpu_sc as plsc

_CHUNK = 80
_NSUB = 5
_SLAB = _CHUNK * _NSUB
_LANES = 16


@functools.lru_cache(maxsize=None)
def _build_gather(n, k, d):
    n_rows = n * k
    slab8 = 8 * k
    info = plsc.get_sparse_core_info()
    nw = info.num_cores * info.num_subcores
    assert n_rows % nw == 0
    b_per_w = n_rows // nw
    assert b_per_w % _SLAB == 0 and _CHUNK % _LANES == 0
    assert slab8 & (slab8 - 1) == 0
    n_slabs = b_per_w // _SLAB
    assert n_slabs % 2 == 1
    g_load = (b_per_w // slab8 + 2) * slab8

    mesh = plsc.VectorSubcoreMesh(core_axis_name="c", subcore_axis_name="s")

    @functools.partial(
        pl.kernel,
        mesh=mesh,
        out_type=jax.ShapeDtypeStruct((n_rows, d), jnp.float32),
        scratch_types=[
            pltpu.VMEM((g_load,), jnp.int32),
            pltpu.VMEM((_NSUB, _CHUNK), jnp.int32),
            pltpu.VMEM((_NSUB, _CHUNK), jnp.int32),
            pltpu.VMEM((_SLAB, d), jnp.float32),
            pltpu.VMEM((_SLAB, d), jnp.float32),
            pltpu.SemaphoreType.DMA,
            pltpu.SemaphoreType.DMA,
            pltpu.SemaphoreType.DMA,
            pltpu.SemaphoreType.DMA,
        ],
        compiler_params=pltpu.CompilerParams(needs_layout_passes=False),
    )
    def gather_k(table_hbm, idx_hbm, out_hbm, gsrc, ix0, ix1, bf0, bf1,
                 g0, g1, w0, w1):
        idxbs = (ix0, ix1)
        bufs = (bf0, bf1)
        gsem = (g0, g1)
        wsem = (w0, w1)
        wid = lax.axis_index("s") * info.num_cores + lax.axis_index("c")
        p0 = wid * b_per_w
        off = jnp.minimum((p0 // slab8) * slab8, n_rows - g_load)
        pltpu.sync_copy(idx_hbm.at[pl.ds(off, g_load)], gsrc)

        def fill_idx(s, par):
            for g in range(_SLAB // _LANES):
                p_vec = (p0 + s * _SLAB + g * _LANES) + lax.iota(jnp.int32, _LANES)
                t = p_vec & (slab8 - 1)
                src = (p_vec - t - off) + (t & 7) * k + (t >> 3)
                idxbs[par][g // (_CHUNK // _LANES),
                           pl.ds(g % (_CHUNK // _LANES) * _LANES, _LANES)] = (
                               plsc.load_gather(gsrc, [src]))

        def gather_descs(par):
            return [
                pltpu.make_async_copy(
                    table_hbm.at[idxbs[par].at[u]],
                    bufs[par].at[pl.ds(u * _CHUNK, _CHUNK)],
                    gsem[par],
                )
                for u in range(_NSUB)
            ]

        def write_desc(s, par):
            return pltpu.make_async_copy(
                bufs[par],
                out_hbm.at[pl.ds(p0 + s * _SLAB, _SLAB)],
                wsem[par],
            )

        def stage(s, par, first):
            if not first:
                @pl.when(s >= 2)
                def _():
                    write_desc(s - 2, par).wait()
            fill_idx(s, par)
            for desc in gather_descs(par):
                desc.start()
            if not first:
                for desc in gather_descs(1 - par):
                    desc.wait()
                write_desc(s - 1, 1 - par).start()

        stage(0, 0, True)

        def body(r, carry):
            stage(2 * r + 1, 1, False)
            stage(2 * r + 2, 0, False)
            return carry

        lax.fori_loop(0, (n_slabs - 1) // 2, body, 0)

        last = n_slabs - 1
        for desc in gather_descs(last & 1):
            desc.wait()
        write_desc(last, last & 1).start()
        write_desc(last - 1, 1 - (last & 1)).wait()
        write_desc(last, last & 1).wait()

    return gather_k


def kernel(logits, G):
    n, d = logits.shape
    k = G.shape[1]
    idx = G.astype(jnp.int32).reshape(-1)
    out = _build_gather(n, k, d)(logits, idx)
    return out.reshape(n // 8, k, 8, d).transpose(0, 2, 1, 3).reshape(n, k * d)
